# baseline (speedup 1.0000x reference)
import functools

import jax
import jax.numpy as jnp
from jax import lax
from jax.experimental import pallas as pl
from jax.experimental.pallas import tpu as pltpu
from jax.experimental.pallas import tpu_sc as plsc

_VOCAB = 1000
_VP = 1024
_NLC = _VP // 128
_TAIL = _VOCAB - 7 * 128
_C = 64
_CP = 128
_T = 8
_B = 4096

_NC = 2
_NS = 16
_NW = _NC * _NS
_ROWS = _B * _T
_RPW = _ROWS // _NW
_CHUNK = 32
_NCHUNK = _RPW // _CHUNK
_NKEY = _T * _VOCAB


def _tt_body(tok_ref, pos_ref, w_ref, b_ref, o_ref):
    t = pl.program_id(0)
    w = w_ref[...]
    p = jnp.dot(pos_ref[pl.ds(t, 1), :], w, preferred_element_type=jnp.float32)
    o_ref[...] = (jnp.dot(tok_ref[...], w, preferred_element_type=jnp.float32)
                  + p + b_ref[...])


def _sc_emit(table, keys4):
    mesh = plsc.VectorSubcoreMesh(core_axis_name="c", subcore_axis_name="s")

    @functools.partial(
        pl.kernel,
        mesh=mesh,
        out_type=jax.ShapeDtypeStruct((_ROWS, _VOCAB), jnp.float32),
        scratch_types=[
            pltpu.VMEM((_NCHUNK * _NLC, _CHUNK), jnp.int32),
            pltpu.VMEM((_NLC, _CHUNK, 128), jnp.float32),
            pltpu.VMEM((_NLC, _CHUNK, 128), jnp.float32),
            pltpu.VMEM((_CHUNK, _TAIL), jnp.float32),
            pltpu.VMEM((_CHUNK, _TAIL), jnp.float32),
            pltpu.SemaphoreType.DMA,
            pltpu.SemaphoreType.DMA,
        ],
    )
    def k(table_hbm, keys_hbm, out_hbm, keys_v, buf_a, buf_b, tl_a, tl_b,
          sem_a, sem_b):
        wid = lax.axis_index("s") * _NC + lax.axis_index("c")
        base = wid * _RPW
        pltpu.sync_copy(keys_hbm.at[wid], keys_v)

        def gathers(m, buf, sem):
            return [pltpu.make_async_copy(
                table_hbm.at[keys_v.at[m * _NLC + c]], buf.at[c], sem)
                for c in range(_NLC)]

        def emit(m, buf, tl):
            r0 = base + m * _CHUNK
            for c in range(_NLC - 1):
                pltpu.sync_copy(buf.at[c],
                                out_hbm.at[pl.ds(r0, _CHUNK), pl.ds(c * 128, 128)])
            for r in range(_CHUNK):
                for l0 in (0, 16, 32, 48, 64, 80, _TAIL - 16):
                    tl[r, pl.ds(l0, 16)] = buf[_NLC - 1, r, pl.ds(l0, 16)]
            pltpu.sync_copy(tl,
                            out_hbm.at[pl.ds(r0, _CHUNK), pl.ds(896, _TAIL)])

        for g in gathers(0, buf_a, sem_a):
            g.start()

        def step(i, _):
            m = 2 * i
            for g in gathers(m, buf_a, sem_a):
                g.wait()
            for g in gathers(m + 1, buf_b, sem_b):
                g.start()
            emit(m, buf_a, tl_a)
            for g in gathers(m + 1, buf_b, sem_b):
                g.wait()

            @pl.when(m + 2 < _NCHUNK)
            def _():
                for g in gathers(m + 2, buf_a, sem_a):
                    g.start()

            emit(m + 1, buf_b, tl_b)
            return _

        lax.fori_loop(0, _NCHUNK // 2, step, None)

    return k(table, keys4)


def kernel(idx, tok_table, pos_table, W, b):
    B, T = idx.shape
    tok_pad = jnp.pad(tok_table, ((0, 0), (0, _CP - _C)))
    pos_pad = jnp.pad(pos_table, ((0, 0), (0, _CP - _C)))
    W_pad = jnp.pad(W, ((0, _CP - _C), (0, _VP - _VOCAB)))
    b_pad = jnp.pad(b, ((0, _VP - _VOCAB),)).reshape(1, _VP)

    table = pl.pallas_call(
        _tt_body,
        grid=(_T, _NLC),
        in_specs=[
            pl.BlockSpec((_VOCAB, _CP), lambda t, c: (0, 0)),
            pl.BlockSpec((_T, _CP), lambda t, c: (0, 0)),
            pl.BlockSpec((_CP, 128), lambda t, c: (0, c)),
            pl.BlockSpec((1, 128), lambda t, c: (0, c)),
        ],
        out_specs=pl.BlockSpec((_VOCAB, 128), lambda t, c: (c * _T + t, 0)),
        out_shape=jax.ShapeDtypeStruct((_NLC * _NKEY, 128), jnp.float32),
    )(tok_pad, pos_pad, W_pad, b_pad)

    kbase = idx + _VOCAB * jnp.arange(_T, dtype=jnp.int32)[None, :]
    keys4 = (kbase.reshape(_NW, _NCHUNK, 1, _CHUNK)
             + _NKEY * jnp.arange(_NLC, dtype=jnp.int32).reshape(1, 1, _NLC, 1)
             ).reshape(_NW, _NCHUNK * _NLC, _CHUNK)

    out = _sc_emit(table, keys4)
    return out.reshape(B, T, _VOCAB)
